# baseline (speedup 1.0000x reference)
import functools

import jax
import jax.numpy as jnp
from jax.experimental import pallas as pl
from jax.experimental.pallas import tpu as pltpu

D = 32
NPB = 128
EPB = 256
CHUNKS = 64
PAIRS = CHUNKS // 2
NPP = 2 * NPB
EPP = 2 * EPB
NB = NPB * CHUNKS
EB = EPB * CHUNKS
ATOM_VOCAB = 24
BOND_VOCAB = 8


def _gin_kernel(nf_ref,
                ed_ref,
                ds_ref,
                at_ref,
                bt_ref,
                w_ref,
                out_ref,
                *, num_layer):
    f32 = jnp.float32
    bf16 = jnp.bfloat16

    def onehot(mask):
        return jnp.where(mask, f32(1.0), f32(0.0)).astype(bf16)

    nfp = nf_ref[0]
    s0 = nfp & 255
    s1 = (nfp >> 8) & 255
    s2 = nfp >> 16
    edp = ed_ref[0]
    src = edp & 127
    ea0 = (edp >> 14) & 3
    ea1 = edp >> 16

    vi = jax.lax.broadcasted_iota(jnp.int32, (ATOM_VOCAB, NB), 0)
    a_m = (vi == s0) | (vi == s1 + 16) | (vi == s2 + 20)
    h = jnp.dot(at_ref[...].astype(bf16), onehot(a_m),
                preferred_element_type=f32)

    bi = jax.lax.broadcasted_iota(jnp.int32, (BOND_VOCAB, EB), 0)
    boh = onehot((bi == ea0) | (bi == ea1 + 4))

    n_iota = jax.lax.broadcasted_iota(jnp.int32, (NPB, EPB), 0)
    e_iota = jax.lax.broadcasted_iota(jnp.int32, (EPB, NPB), 1)
    zg = jnp.zeros((NPB, EPB), bf16)
    zs = jnp.zeros((EPB, NPB), bf16)
    sts, dohs = [], []
    for p in range(PAIRS):
        s0 = onehot(n_iota == src[0:1, p * EPP:p * EPP + EPB])
        s1 = onehot(n_iota == src[0:1, p * EPP + EPB:(p + 1) * EPP])
        sts.append(jnp.concatenate(
            [jnp.concatenate([s0, zg], axis=1),
             jnp.concatenate([zg, s1], axis=1)], axis=0))
        d0 = onehot(e_iota == ds_ref[0, 0:EPB, p:p + 1])
        d1 = onehot(e_iota == ds_ref[0, EPB:EPP, p:p + 1])
        dohs.append(jnp.concatenate(
            [jnp.concatenate([d0, zs], axis=1),
             jnp.concatenate([zs, d1], axis=1)], axis=0))

    for l in range(num_layer):
        w1t = w_ref[l, :, 0:D].astype(bf16)
        b1t = w_ref[l, :, D:D + 1]
        ope = w_ref[l, 0:D, D + 1:D + 2]
        b2t = w_ref[l, 0:D, D + 2:D + 3]
        w2t = w_ref[l, 0:D, 2 * D:4 * D].astype(bf16)

        eemb = jnp.dot(bt_ref[l].astype(bf16), boh,
                       preferred_element_type=f32)

        hb = h.astype(bf16)
        x_js = [jnp.dot(hb[:, p * NPP:(p + 1) * NPP], sts[p],
                        preferred_element_type=f32)
                for p in range(PAIRS)]
        msgs = [jnp.maximum(x_js[p] + eemb[:, p * EPP:(p + 1) * EPP],
                            0.0).astype(bf16)
                for p in range(PAIRS)]
        aggs = [jnp.dot(msgs[p], dohs[p], preferred_element_type=f32)
                for p in range(PAIRS)]
        agg = jnp.concatenate(aggs, axis=1)

        z = (ope * h + agg).astype(bf16)
        h1 = jnp.maximum(jnp.dot(w1t, z, preferred_element_type=f32)
                         + b1t, 0.0).astype(bf16)
        h = jnp.dot(w2t, h1, preferred_element_type=f32) + b2t
        if l < num_layer - 1:
            h = jnp.maximum(h, 0.0)

    out_ref[...] = jnp.transpose(h.reshape(D, NB // 128, 128), (1, 0, 2))


def kernel(node_feats, edge_index, edge_attr,
           atom_tables_0, atom_tables_1, atom_tables_2,
           bond_tables_0_0, bond_tables_0_1, bond_tables_1_0, bond_tables_1_1,
           wslab):
    total_n = node_feats.shape[0]
    total_e = edge_index.shape[1]
    num_layer = wslab.shape[0]
    num_steps = total_n // NB

    nf = node_feats.astype(jnp.int32)
    nfp = (nf[:, 0] | (nf[:, 1] << 8) | (nf[:, 2] << 16)) \
        .reshape(num_steps, 1, NB)
    ei = edge_index.astype(jnp.int32)
    ea = edge_attr.astype(jnp.int32)
    edp = ((ei[0] & (NPB - 1))
           | (ea[:, 0] << 14) | (ea[:, 1] << 16)) \
        .reshape(num_steps, 1, EB)
    dsub = (ei[1] & (NPB - 1)).reshape(num_steps, PAIRS, EPP) \
        .swapaxes(1, 2)

    atomT = jnp.concatenate([atom_tables_0, atom_tables_1, atom_tables_2],
                            axis=0).T
    pad = jnp.zeros((1, D), jnp.float32)
    btT = jnp.stack([
        jnp.concatenate([bond_tables_0_0, bond_tables_0_1, pad], axis=0).T,
        jnp.concatenate([bond_tables_1_0, bond_tables_1_1, pad], axis=0).T,
    ], axis=0)

    body = functools.partial(_gin_kernel, num_layer=num_layer)

    flops_step = num_layer * PAIRS * (2 * D * NPP * EPP * 2) \
        + num_layer * (2 * 2 * D * D * NB * 2 + 2 * D * BOND_VOCAB * EB) \
        + 2 * D * ATOM_VOCAB * NB
    bytes_step = 4 * (NB + EB + NB * D)

    grid_spec = pltpu.PrefetchScalarGridSpec(
        num_scalar_prefetch=0,
        grid=(num_steps,),
        in_specs=[
            pl.BlockSpec((1, 1, NB), lambda i: (i, 0, 0)),
            pl.BlockSpec((1, 1, EB), lambda i: (i, 0, 0)),
            pl.BlockSpec((1, EPP, PAIRS), lambda i: (i, 0, 0)),
            pl.BlockSpec((D, ATOM_VOCAB), lambda i: (0, 0)),
            pl.BlockSpec((num_layer, D, BOND_VOCAB), lambda i: (0, 0, 0)),
            pl.BlockSpec((num_layer, 2 * D, 4 * D), lambda i: (0, 0, 0)),
        ],
        out_specs=pl.BlockSpec((NB // 128, D, 128), lambda i: (i, 0, 0)),
    )
    outT = pl.pallas_call(
        body,
        out_shape=jax.ShapeDtypeStruct((total_n // 128, D, 128), jnp.float32),
        grid_spec=grid_spec,
        compiler_params=pltpu.CompilerParams(dimension_semantics=("parallel",)),
        cost_estimate=pl.CostEstimate(flops=num_steps * flops_step,
                                      transcendentals=0,
                                      bytes_accessed=num_steps * bytes_step),
    )(nfp, edp, dsub, atomT, btT, wslab)
    return outT.transpose(0, 2, 1).reshape(total_n, D)


# speedup vs baseline: 1.0327x; 1.0327x over previous
import functools

import jax
import jax.numpy as jnp
from jax.experimental import pallas as pl
from jax.experimental.pallas import tpu as pltpu

D = 32
NPB = 128
EPB = 256
CHUNKS = 64
PAIRS = CHUNKS // 2
NPP = 2 * NPB
EPP = 2 * EPB
NB = NPB * CHUNKS
EB = EPB * CHUNKS
ATOM_VOCAB = 24
BOND_VOCAB = 8


def _gin_kernel(nf_ref,
                ed_ref,
                ds_ref,
                at_ref,
                wb_ref,
                w_ref,
                out_ref,
                *, num_layer):
    f32 = jnp.float32
    bf16 = jnp.bfloat16

    def onehot(mask):
        return jnp.where(mask, f32(1.0), f32(0.0)).astype(bf16)

    nfp = nf_ref[0]
    s0 = nfp & 255
    s1 = (nfp >> 8) & 255
    s2 = nfp >> 16
    edp = ed_ref[0]
    src = edp & 127
    ea0 = (edp >> 14) & 3
    ea1 = edp >> 16

    vi = jax.lax.broadcasted_iota(jnp.int32, (ATOM_VOCAB, NB), 0)
    a_m = (vi == s0) | (vi == s1 + 16) | (vi == s2 + 20)
    h = jnp.dot(at_ref[...].astype(bf16), onehot(a_m),
                preferred_element_type=f32)

    bi = jax.lax.broadcasted_iota(jnp.int32, (BOND_VOCAB, EB), 0)
    boh = onehot((bi == ea0) | (bi == ea1 + 4))

    n_iota = jax.lax.broadcasted_iota(jnp.int32, (NPB, EPB), 0)
    e_iota = jax.lax.broadcasted_iota(jnp.int32, (EPB, NPB), 1)
    zg = jnp.zeros((NPB, EPB), bf16)
    zs = jnp.zeros((EPB, NPB), bf16)
    sts, dohs = [], []
    for p in range(PAIRS):
        s0 = onehot(n_iota == src[0:1, p * EPP:p * EPP + EPB])
        s1 = onehot(n_iota == src[0:1, p * EPP + EPB:(p + 1) * EPP])
        sts.append(jnp.concatenate(
            [jnp.concatenate([s0, zg], axis=1),
             jnp.concatenate([zg, s1], axis=1)], axis=0))
        d0 = onehot(e_iota == ds_ref[0, 0:EPB, p:p + 1])
        d1 = onehot(e_iota == ds_ref[0, EPB:EPP, p:p + 1])
        dohs.append(jnp.concatenate(
            [jnp.concatenate([d0, zs], axis=1),
             jnp.concatenate([zs, d1], axis=1)], axis=0))

    hb = h.astype(bf16)
    for l in range(num_layer):
        b1t = w_ref[l, :, D:D + 1]
        b2t = w_ref[l, 0:D, D + 2:D + 3]
        w1c = wb_ref[l, :, 0:2 * D]
        w2t = wb_ref[l, 0:D, 2 * D:4 * D]
        btb = wb_ref[l, 0:D, 4 * D:4 * D + BOND_VOCAB]

        x_js = [jnp.dot(
            jnp.concatenate([hb[:, p * NPP:(p + 1) * NPP], btb], axis=1),
            jnp.concatenate([sts[p], boh[:, p * EPP:(p + 1) * EPP]], axis=0),
            preferred_element_type=f32)
            for p in range(PAIRS)]
        msgs = [jnp.maximum(x_js[p], 0.0).astype(bf16)
                for p in range(PAIRS)]
        aggs = [jnp.dot(msgs[p], dohs[p], preferred_element_type=f32)
                for p in range(PAIRS)]
        agg = jnp.concatenate(aggs, axis=1).astype(bf16)

        zcat = jnp.concatenate([agg, hb], axis=0)
        h1 = jnp.maximum(jnp.dot(w1c, zcat, preferred_element_type=f32)
                         + b1t, 0.0).astype(bf16)
        h = jnp.dot(w2t, h1, preferred_element_type=f32) + b2t
        if l < num_layer - 1:
            h = jnp.maximum(h, 0.0)
            hb = h.astype(bf16)

    out_ref[...] = h


def kernel(node_feats, edge_index, edge_attr,
           atom_tables_0, atom_tables_1, atom_tables_2,
           bond_tables_0_0, bond_tables_0_1, bond_tables_1_0, bond_tables_1_1,
           wslab):
    total_n = node_feats.shape[0]
    total_e = edge_index.shape[1]
    num_layer = wslab.shape[0]
    num_steps = total_n // NB

    nf = node_feats.astype(jnp.int32)
    nfp = (nf[:, 0] | (nf[:, 1] << 8) | (nf[:, 2] << 16)) \
        .reshape(num_steps, 1, NB)
    ei = edge_index.astype(jnp.int32)
    ea = edge_attr.astype(jnp.int32)
    edp = ((ei[0] & (NPB - 1))
           | (ea[:, 0] << 14) | (ea[:, 1] << 16)) \
        .reshape(num_steps, 1, EB)
    dsub = (ei[1] & (NPB - 1)).reshape(num_steps, PAIRS, EPP) \
        .swapaxes(1, 2)

    atomT = jnp.concatenate([atom_tables_0, atom_tables_1, atom_tables_2],
                            axis=0).T
    pad = jnp.zeros((1, D), jnp.float32)
    btT = jnp.stack([
        jnp.concatenate([bond_tables_0_0, bond_tables_0_1, pad], axis=0).T,
        jnp.concatenate([bond_tables_1_0, bond_tables_1_1, pad], axis=0).T,
    ], axis=0)
    wb = jnp.zeros((num_layer, 2 * D, 4 * D + BOND_VOCAB), jnp.float32)
    for l in range(num_layer):
        w1t = wslab[l, :, 0:D]
        ope = wslab[l, 0:D, D + 1]
        wb = wb.at[l, :, 0:D].set(w1t)
        wb = wb.at[l, :, D:2 * D].set(w1t * ope[None, :])
        wb = wb.at[l, 0:D, 2 * D:4 * D].set(wslab[l, 0:D, 2 * D:4 * D])
        wb = wb.at[l, 0:D, 4 * D:].set(btT[l])
    wb = wb.astype(jnp.bfloat16)

    body = functools.partial(_gin_kernel, num_layer=num_layer)

    flops_step = num_layer * PAIRS * (2 * D * NPP * EPP * 2) \
        + num_layer * (2 * 2 * D * D * NB * 2 + 2 * D * BOND_VOCAB * EB) \
        + 2 * D * ATOM_VOCAB * NB
    bytes_step = 4 * (NB + EB + NB * D)

    grid_spec = pltpu.PrefetchScalarGridSpec(
        num_scalar_prefetch=0,
        grid=(num_steps,),
        in_specs=[
            pl.BlockSpec((1, 1, NB), lambda i: (i, 0, 0)),
            pl.BlockSpec((1, 1, EB), lambda i: (i, 0, 0)),
            pl.BlockSpec((1, EPP, PAIRS), lambda i: (i, 0, 0)),
            pl.BlockSpec((D, ATOM_VOCAB), lambda i: (0, 0)),
            pl.BlockSpec((num_layer, 2 * D, 4 * D + BOND_VOCAB),
                         lambda i: (0, 0, 0)),
            pl.BlockSpec((num_layer, 2 * D, 4 * D), lambda i: (0, 0, 0)),
        ],
        out_specs=pl.BlockSpec((D, NB), lambda i: (0, i)),
    )
    outT = pl.pallas_call(
        body,
        out_shape=jax.ShapeDtypeStruct((D, total_n), jnp.float32),
        grid_spec=grid_spec,
        compiler_params=pltpu.CompilerParams(dimension_semantics=("parallel",)),
        cost_estimate=pl.CostEstimate(flops=num_steps * flops_step,
                                      transcendentals=0,
                                      bytes_accessed=num_steps * bytes_step),
    )(nfp, edp, dsub, atomT, wb, wslab)
    return outT.T


# speedup vs baseline: 1.1137x; 1.0784x over previous
import functools

import jax
import jax.numpy as jnp
from jax.experimental import pallas as pl
from jax.experimental.pallas import tpu as pltpu

D = 32
NPB = 128
EPB = 256
CHUNKS = 64
PAIRS = CHUNKS // 2
NPP = 2 * NPB
EPP = 2 * EPB
NB = NPB * CHUNKS
EB = EPB * CHUNKS
ATOM_VOCAB = 24
BOND_VOCAB = 8


def _gin_kernel(nf_ref,
                ed_ref,
                ds_ref,
                at_ref,
                bt_ref,
                w_ref,
                out_ref,
                *, num_layer):
    f32 = jnp.float32
    bf16 = jnp.bfloat16

    def onehot(mask):
        return jnp.where(mask, f32(1.0), f32(0.0)).astype(bf16)

    nfp = nf_ref[0]
    s0 = nfp & 255
    s1 = (nfp >> 8) & 255
    s2 = nfp >> 16
    edp = ed_ref[0]
    src = edp & 127
    ea0 = (edp >> 14) & 3
    ea1 = edp >> 16

    vi = jax.lax.broadcasted_iota(jnp.int32, (ATOM_VOCAB, NB), 0)
    a_m = (vi == s0) | (vi == s1 + 16) | (vi == s2 + 20)
    h = jnp.dot(at_ref[...].astype(bf16), onehot(a_m),
                preferred_element_type=f32)

    bi = jax.lax.broadcasted_iota(jnp.int32, (BOND_VOCAB, EB), 0)
    boh = onehot((bi == ea0) | (bi == ea1 + 4))

    n_iota = jax.lax.broadcasted_iota(jnp.int32, (NPB, EPB), 0)
    e_iota = jax.lax.broadcasted_iota(jnp.int32, (EPB, NPB), 1)
    zg = jnp.zeros((NPB, EPB), bf16)
    zs = jnp.zeros((EPB, NPB), bf16)
    sts, dohs = [], []
    for p in range(PAIRS):
        s0 = onehot(n_iota == src[0:1, p * EPP:p * EPP + EPB])
        s1 = onehot(n_iota == src[0:1, p * EPP + EPB:(p + 1) * EPP])
        sts.append(jnp.concatenate(
            [jnp.concatenate([s0, zg], axis=1),
             jnp.concatenate([zg, s1], axis=1)], axis=0))
        d0 = onehot(e_iota == ds_ref[0, 0:EPB, p:p + 1])
        d1 = onehot(e_iota == ds_ref[0, EPB:EPP, p:p + 1])
        dohs.append(jnp.concatenate(
            [jnp.concatenate([d0, zs], axis=1),
             jnp.concatenate([zs, d1], axis=1)], axis=0))

    for l in range(num_layer):
        w1t = w_ref[l, :, 0:D].astype(bf16)
        b1t = w_ref[l, :, D:D + 1]
        ope = w_ref[l, 0:D, D + 1:D + 2]
        b2t = w_ref[l, 0:D, D + 2:D + 3]
        w2t = w_ref[l, 0:D, 2 * D:4 * D].astype(bf16)

        eemb = jnp.dot(bt_ref[l].astype(bf16), boh,
                       preferred_element_type=f32)

        hb = h.astype(bf16)
        x_js = [jnp.dot(hb[:, p * NPP:(p + 1) * NPP], sts[p],
                        preferred_element_type=f32)
                for p in range(PAIRS)]
        msgs = [jnp.maximum(x_js[p] + eemb[:, p * EPP:(p + 1) * EPP],
                            0.0).astype(bf16)
                for p in range(PAIRS)]
        aggs = [jnp.dot(msgs[p], dohs[p], preferred_element_type=f32)
                for p in range(PAIRS)]
        agg = jnp.concatenate(aggs, axis=1)

        z = (ope * h + agg).astype(bf16)
        h1 = jnp.maximum(jnp.dot(w1t, z, preferred_element_type=f32)
                         + b1t, 0.0).astype(bf16)
        h = jnp.dot(w2t, h1, preferred_element_type=f32) + b2t
        if l < num_layer - 1:
            h = jnp.maximum(h, 0.0)

    out_ref[...] = h


def kernel(node_feats, edge_index, edge_attr,
           atom_tables_0, atom_tables_1, atom_tables_2,
           bond_tables_0_0, bond_tables_0_1, bond_tables_1_0, bond_tables_1_1,
           wslab):
    total_n = node_feats.shape[0]
    total_e = edge_index.shape[1]
    num_layer = wslab.shape[0]
    num_steps = total_n // NB

    nf = node_feats.astype(jnp.int32)
    nfp = (nf[:, 0] | (nf[:, 1] << 8) | (nf[:, 2] << 16)) \
        .reshape(num_steps, 1, NB)
    ei = edge_index.astype(jnp.int32)
    ea = edge_attr.astype(jnp.int32)
    edp = ((ei[0] & (NPB - 1))
           | (ea[:, 0] << 14) | (ea[:, 1] << 16)) \
        .reshape(num_steps, 1, EB)
    dsub = (ei[1] & (NPB - 1)).reshape(num_steps, PAIRS, EPP) \
        .swapaxes(1, 2)

    atomT = jnp.concatenate([atom_tables_0, atom_tables_1, atom_tables_2],
                            axis=0).T
    pad = jnp.zeros((1, D), jnp.float32)
    btT = jnp.stack([
        jnp.concatenate([bond_tables_0_0, bond_tables_0_1, pad], axis=0).T,
        jnp.concatenate([bond_tables_1_0, bond_tables_1_1, pad], axis=0).T,
    ], axis=0)

    body = functools.partial(_gin_kernel, num_layer=num_layer)

    flops_step = num_layer * PAIRS * (2 * D * NPP * EPP * 2) \
        + num_layer * (2 * 2 * D * D * NB * 2 + 2 * D * BOND_VOCAB * EB) \
        + 2 * D * ATOM_VOCAB * NB
    bytes_step = 4 * (NB + EB + NB * D)

    grid_spec = pltpu.PrefetchScalarGridSpec(
        num_scalar_prefetch=0,
        grid=(num_steps,),
        in_specs=[
            pl.BlockSpec((1, 1, NB), lambda i: (i, 0, 0)),
            pl.BlockSpec((1, 1, EB), lambda i: (i, 0, 0)),
            pl.BlockSpec((1, EPP, PAIRS), lambda i: (i, 0, 0)),
            pl.BlockSpec((D, ATOM_VOCAB), lambda i: (0, 0)),
            pl.BlockSpec((num_layer, D, BOND_VOCAB), lambda i: (0, 0, 0)),
            pl.BlockSpec((num_layer, 2 * D, 4 * D), lambda i: (0, 0, 0)),
        ],
        out_specs=pl.BlockSpec((D, NB), lambda i: (0, i)),
    )
    outT = pl.pallas_call(
        body,
        out_shape=jax.ShapeDtypeStruct((D, total_n), jnp.float32),
        grid_spec=grid_spec,
        compiler_params=pltpu.CompilerParams(dimension_semantics=("parallel",)),
        cost_estimate=pl.CostEstimate(flops=num_steps * flops_step,
                                      transcendentals=0,
                                      bytes_accessed=num_steps * bytes_step),
    )(nfp, edp, dsub, atomT, btT, wslab)
    return outT.T
